# Initial kernel scaffold; baseline (speedup 1.0000x reference)
import jax
import jax.numpy as jnp
from jax.experimental import pallas as pl


def kernel(h, e, edge_index, Wh, bh, We, be, Wl, bl, gamma, beta, W1, b1, W2, b2, W3, b3):
    raise NotImplementedError("write your pallas kernel here")


# speedup vs baseline: 1.8796x; 1.8796x over previous
import functools

import jax
import jax.numpy as jnp
from jax import lax
from jax.experimental import pallas as pl
from jax.experimental.pallas import tpu as pltpu
from jax.experimental.pallas import tpu_sc as plsc

N = 10000
E = 320000
D = 128
H = 64
L = 4
NC = 10
C = 125
NSUB = 16
EPT = E // NSUB
NCH = EPT // C
NPT = N // NSUB
BE = 3200
F32 = jnp.float32


def _pe_body(h_ref, w_ref, b_ref, o_ref):
    o_ref[...] = jnp.dot(h_ref[...], w_ref[...],
                         preferred_element_type=F32) + b_ref[...]


def _pe(h, Wh, bh):
    return pl.pallas_call(
        _pe_body,
        out_shape=jax.ShapeDtypeStruct((N, D), F32),
    )(h, Wh, bh.reshape(1, D))


def _node_mm_body(h_ref, w0, w1, w3, w4, b0, b1, b3, b4,
                  ah_ref, blo, bhi, dlo, dhi, elo, ehi):
    x = h_ref[...]
    A = jnp.dot(x, w0[...], preferred_element_type=F32) + b0[...]
    B = jnp.dot(x, w1[...], preferred_element_type=F32) + b1[...]
    Dm = jnp.dot(x, w3[...], preferred_element_type=F32) + b3[...]
    Em = jnp.dot(x, w4[...], preferred_element_type=F32) + b4[...]
    ah_ref[...] = A
    blo[...] = B[:, :H]
    bhi[...] = B[:, H:]
    dlo[...] = Dm[:, :H]
    dhi[...] = Dm[:, H:]
    elo[...] = Em[:, :H]
    ehi[...] = Em[:, H:]


def _node_mm(h, Wl_l, bl_l):
    outs = [jax.ShapeDtypeStruct((N, D), F32)] + \
           [jax.ShapeDtypeStruct((N, H), F32)] * 6
    return pl.pallas_call(
        _node_mm_body,
        out_shape=outs,
    )(h, Wl_l[0], Wl_l[1], Wl_l[3], Wl_l[4],
      bl_l[0].reshape(1, D), bl_l[1].reshape(1, D),
      bl_l[3].reshape(1, D), bl_l[4].reshape(1, D))


def _edgemm_body(eflo, efhi, w_ref, b_ref, clo, chi):
    w = w_ref[...]
    y = (jnp.dot(eflo[...], w[:H], preferred_element_type=F32)
         + jnp.dot(efhi[...], w[H:], preferred_element_type=F32)
         + b_ref[...])
    clo[...] = y[:, :H]
    chi[...] = y[:, H:]


def _edgemm(eflo, efhi, W2, b2):
    return pl.pallas_call(
        _edgemm_body,
        grid=(E // BE,),
        in_specs=[pl.BlockSpec((BE, H), lambda i: (i, 0)),
                  pl.BlockSpec((BE, H), lambda i: (i, 0)),
                  pl.BlockSpec((D, D), lambda i: (0, 0)),
                  pl.BlockSpec((1, D), lambda i: (0, 0))],
        out_specs=[pl.BlockSpec((BE, H), lambda i: (i, 0)),
                   pl.BlockSpec((BE, H), lambda i: (i, 0))],
        out_shape=[jax.ShapeDtypeStruct((E, H), F32)] * 2,
    )(eflo, efhi, W2, b2.reshape(1, D))


def _edgebn_body(enlo, enhi, eflo, efhi, ssum, ssq, g_ref, bt_ref, olo, ohi):
    ss = ssum[...]
    qq = ssq[...]
    mlo = jnp.sum(ss[:NSUB], axis=0, keepdims=True) * (1.0 / E)
    mhi = jnp.sum(ss[NSUB:], axis=0, keepdims=True) * (1.0 / E)
    vlo = jnp.sum(qq[:NSUB], axis=0, keepdims=True) * (1.0 / E) - mlo * mlo
    vhi = jnp.sum(qq[NSUB:], axis=0, keepdims=True) * (1.0 / E) - mhi * mhi
    rlo = lax.rsqrt(vlo + 1e-5)
    rhi = lax.rsqrt(vhi + 1e-5)
    g = g_ref[...]
    bt = bt_ref[...]
    olo[...] = jnp.maximum((enlo[...] - mlo) * rlo * g[:, :H] + bt[:, :H],
                           0.0) + eflo[...]
    ohi[...] = jnp.maximum((enhi[...] - mhi) * rhi * g[:, H:] + bt[:, H:],
                           0.0) + efhi[...]


def _edgebn(enlo, enhi, eflo, efhi, ssum, ssq, g, bt):
    eb = pl.BlockSpec((BE, H), lambda i: (i, 0))
    return pl.pallas_call(
        _edgebn_body,
        grid=(E // BE,),
        in_specs=[eb, eb, eb, eb,
                  pl.BlockSpec((2 * NSUB, H), lambda i: (0, 0)),
                  pl.BlockSpec((2 * NSUB, H), lambda i: (0, 0)),
                  pl.BlockSpec((1, D), lambda i: (0, 0)),
                  pl.BlockSpec((1, D), lambda i: (0, 0))],
        out_specs=[eb, eb],
        out_shape=[jax.ShapeDtypeStruct((E, H), F32)] * 2,
    )(enlo, enhi, eflo, efhi, ssum, ssq, g.reshape(1, D), bt.reshape(1, D))


def _h_update(ah, nlo, nhi, dlo, dhi, hin, g, bt):
    num = jnp.concatenate([nlo, nhi], axis=1)
    den = jnp.concatenate([dlo, dhi], axis=1)
    x = ah + num / (den + 1e-6)
    mu = jnp.mean(x, axis=0, keepdims=True)
    var = jnp.mean(x * x, axis=0, keepdims=True) - mu * mu
    x = (x - mu) * lax.rsqrt(var + 1e-5) * g + bt
    return jnp.maximum(x, 0.0) + hin


def _hpost_body(ah, nlo, nhi, dlo, dhi, hin, g, bt, o_ref):
    o_ref[...] = _h_update(ah[...], nlo[...], nhi[...], dlo[...], dhi[...],
                           hin[...], g[...], bt[...])


def _hpost(ah, nlo, nhi, dlo, dhi, hin, g, bt):
    return pl.pallas_call(
        _hpost_body,
        out_shape=jax.ShapeDtypeStruct((N, D), F32),
    )(ah, nlo, nhi, dlo, dhi, hin, g.reshape(1, D), bt.reshape(1, D))


def _hlast_body(ah, nlo, nhi, dlo, dhi, hin, g, bt,
                w1, b1, w2, b2, w3, b3, o_ref):
    hf = _h_update(ah[...], nlo[...], nhi[...], dlo[...], dhi[...],
                   hin[...], g[...], bt[...])
    hg = jnp.mean(hf, axis=0, keepdims=True)
    t = jnp.maximum(jnp.dot(hg, w1[...], preferred_element_type=F32)
                    + b1[...], 0.0)
    t = jnp.maximum(jnp.dot(t, w2[...], preferred_element_type=F32)
                    + b2[...], 0.0)
    o_ref[...] = jnp.dot(t, w3[...], preferred_element_type=F32) + b3[...]


def _hlast(ah, nlo, nhi, dlo, dhi, hin, g, bt, W1, b1, W2, b2, W3, b3):
    return pl.pallas_call(
        _hlast_body,
        out_shape=jax.ShapeDtypeStruct((1, NC), F32),
    )(ah, nlo, nhi, dlo, dhi, hin, g.reshape(1, D), bt.reshape(1, D),
      W1, b1.reshape(1, D // 2), W2, b2.reshape(1, D // 4),
      W3, b3.reshape(1, NC))


def _sc_edge_body(src2, dst2, blo, bhi, dlo, dhi, elo, ehi, clo, chi,
                  enlo, enhi, nlo, nhi, delo, dehi, ssum, ssq,
                  srcv, dstv, bbuf, dbuf, ebuf, cbuf, sbuf, nbuf, statv,
                  nacc, dacc, sem):
    c = lax.axis_index("c")
    s = lax.axis_index("s")
    z16 = jnp.zeros((16,), F32)

    for kk in range(4):
        statv[0, pl.ds(kk * 16, 16)] = z16
        statv[1, pl.ds(kk * 16, 16)] = z16

    def zrow(r, carry):
        for kk in range(4):
            cbuf[r, pl.ds(kk * 16, 16)] = z16
        return carry

    lax.fori_loop(0, C, zrow, 0)
    for j in range(NPT // C):
        sl = pl.ds(s * NPT + j * C, C)
        pltpu.sync_copy(cbuf, nacc.at[sl])
        pltpu.sync_copy(cbuf, dacc.at[sl])
    plsc.subcore_barrier()

    def do_half(tb, td, te, tc, eno, nmo, dno, srow):
        def chunk(i, carry):
            r = s * NCH + i
            e0 = r * C
            pltpu.sync_copy(src2.at[r], srcv)
            pltpu.sync_copy(dst2.at[r], dstv)
            pltpu.async_copy(tb.at[srcv], bbuf, sem).wait()
            pltpu.async_copy(td.at[srcv], dbuf, sem).wait()
            pltpu.async_copy(te.at[dstv], ebuf, sem).wait()
            pltpu.sync_copy(tc.at[pl.ds(e0, C)], cbuf)

            def rowfn(rr, acc):
                out = []
                for kk in range(4):
                    sl = pl.ds(kk * 16, 16)
                    x = cbuf[rr, sl] + dbuf[rr, sl] + ebuf[rr, sl]
                    cbuf[rr, sl] = x
                    sg = 1.0 / (1.0 + jnp.exp(-x))
                    sbuf[rr, sl] = sg
                    nbuf[rr, sl] = sg * bbuf[rr, sl]
                    out.append((acc[kk] + x, acc[kk + 4] + x * x))
                return tuple(a for a, _ in out) + tuple(b for _, b in out)

            acc = lax.fori_loop(0, C, rowfn,
                                tuple(jnp.zeros((16,), F32) for _ in range(8)))
            for kk in range(4):
                sl = pl.ds(kk * 16, 16)
                statv[0, sl] += acc[kk]
                statv[1, sl] += acc[kk + 4]
            pltpu.sync_copy(cbuf, eno.at[pl.ds(e0, C)])
            pltpu.sync_copy(nbuf, nacc.at[dstv], add=True)
            pltpu.sync_copy(sbuf, dacc.at[dstv], add=True)
            return carry

        lax.fori_loop(0, NCH, chunk, 0)
        plsc.subcore_barrier()
        for j in range(NPT // C):
            sl = pl.ds(s * NPT + j * C, C)
            pltpu.sync_copy(nacc.at[sl], nmo.at[sl])
            pltpu.sync_copy(dacc.at[sl], dno.at[sl])
        pltpu.sync_copy(statv.at[pl.ds(0, 1)], ssum.at[pl.ds(srow + s, 1)])
        pltpu.sync_copy(statv.at[pl.ds(1, 1)], ssq.at[pl.ds(srow + s, 1)])

    @pl.when(c == 0)
    def _():
        do_half(blo, dlo, elo, clo, enlo, nlo, delo, 0)

    @pl.when(c == 1)
    def _():
        do_half(bhi, dhi, ehi, chi, enhi, nhi, dehi, NSUB)


def _sc_edge(src2, dst2, blo, bhi, dlo, dhi, elo, ehi, clo, chi):
    mesh = plsc.VectorSubcoreMesh(core_axis_name="c", subcore_axis_name="s",
                                  num_cores=2, num_subcores=NSUB)
    out_type = [jax.ShapeDtypeStruct((E, H), F32),
                jax.ShapeDtypeStruct((E, H), F32),
                jax.ShapeDtypeStruct((N, H), F32),
                jax.ShapeDtypeStruct((N, H), F32),
                jax.ShapeDtypeStruct((N, H), F32),
                jax.ShapeDtypeStruct((N, H), F32),
                jax.ShapeDtypeStruct((2 * NSUB, H), F32),
                jax.ShapeDtypeStruct((2 * NSUB, H), F32)]
    scratch = [pltpu.VMEM((C,), jnp.int32),
               pltpu.VMEM((C,), jnp.int32),
               pltpu.VMEM((C, H), F32),
               pltpu.VMEM((C, H), F32),
               pltpu.VMEM((C, H), F32),
               pltpu.VMEM((C, H), F32),
               pltpu.VMEM((C, H), F32),
               pltpu.VMEM((C, H), F32),
               pltpu.VMEM((2, H), F32),
               pltpu.VMEM_SHARED((N, H), F32),
               pltpu.VMEM_SHARED((N, H), F32),
               pltpu.SemaphoreType.DMA]
    fn = pl.kernel(_sc_edge_body, out_type=out_type, mesh=mesh,
                   scratch_types=scratch,
                   compiler_params=pltpu.CompilerParams(
                       use_tc_tiling_on_sc=False))
    return fn(src2, dst2, blo, bhi, dlo, dhi, elo, ehi, clo, chi)


def kernel(h, e, edge_index, Wh, bh, We, be, Wl, bl, gamma, beta,
           W1, b1, W2, b2, W3, b3):
    src2 = edge_index[0].reshape(E // C, C)
    dst2 = edge_index[1].reshape(E // C, C)
    hcur = _pe(h, Wh, bh)
    row0 = We[0] + be
    eflo = jnp.broadcast_to(row0[:H], (E, H))
    efhi = jnp.broadcast_to(row0[H:], (E, H))
    logits = None
    for l in range(L):
        ah, blo, bhi, dlo, dhi, elo, ehi = _node_mm(hcur, Wl[l], bl[l])
        clo, chi = _edgemm(eflo, efhi, Wl[l, 2], bl[l, 2])
        enlo, enhi, nlo, nhi, delo, dehi, ssum, ssq = _sc_edge(
            src2, dst2, blo, bhi, dlo, dhi, elo, ehi, clo, chi)
        if l < L - 1:
            hcur = _hpost(ah, nlo, nhi, delo, dehi, hcur,
                          gamma[l, 0], beta[l, 0])
            eflo, efhi = _edgebn(enlo, enhi, eflo, efhi, ssum, ssq,
                                 gamma[l, 1], beta[l, 1])
        else:
            logits = _hlast(ah, nlo, nhi, delo, dehi, hcur,
                            gamma[l, 0], beta[l, 0],
                            W1, b1, W2, b2, W3, b3)
    return logits.reshape(NC)


# speedup vs baseline: 2.2250x; 1.1837x over previous
import functools

import jax
import jax.numpy as jnp
from jax import lax
from jax.experimental import pallas as pl
from jax.experimental.pallas import tpu as pltpu
from jax.experimental.pallas import tpu_sc as plsc

N = 10000
E = 320000
D = 128
H = 64
L = 4
NC = 10
C = 80
NSUB = 16
EPT = E // NSUB
NCH = EPT // C
NPT = N // NSUB
BE = 3200
F32 = jnp.float32


def _pe_body(h_ref, w_ref, b_ref, o_ref):
    o_ref[...] = jnp.dot(h_ref[...], w_ref[...],
                         preferred_element_type=F32) + b_ref[...]


def _pe(h, Wh, bh):
    return pl.pallas_call(
        _pe_body,
        out_shape=jax.ShapeDtypeStruct((N, D), F32),
    )(h, Wh, bh.reshape(1, D))


def _node_mm_body(h_ref, w0, w1, w3, w4, b0, b1, b3, b4,
                  ah_ref, blo, bhi, dlo, dhi, elo, ehi):
    x = h_ref[...]
    A = jnp.dot(x, w0[...], preferred_element_type=F32) + b0[...]
    B = jnp.dot(x, w1[...], preferred_element_type=F32) + b1[...]
    Dm = jnp.dot(x, w3[...], preferred_element_type=F32) + b3[...]
    Em = jnp.dot(x, w4[...], preferred_element_type=F32) + b4[...]
    ah_ref[...] = A
    blo[...] = B[:, :H]
    bhi[...] = B[:, H:]
    dlo[...] = Dm[:, :H]
    dhi[...] = Dm[:, H:]
    elo[...] = Em[:, :H]
    ehi[...] = Em[:, H:]


def _node_mm(h, Wl_l, bl_l):
    outs = [jax.ShapeDtypeStruct((N, D), F32)] + \
           [jax.ShapeDtypeStruct((N, H), F32)] * 6
    return pl.pallas_call(
        _node_mm_body,
        out_shape=outs,
    )(h, Wl_l[0], Wl_l[1], Wl_l[3], Wl_l[4],
      bl_l[0].reshape(1, D), bl_l[1].reshape(1, D),
      bl_l[3].reshape(1, D), bl_l[4].reshape(1, D))


def _edgemm_body(eflo, efhi, w_ref, b_ref, clo, chi):
    w = w_ref[...]
    y = (jnp.dot(eflo[...], w[:H], preferred_element_type=F32)
         + jnp.dot(efhi[...], w[H:], preferred_element_type=F32)
         + b_ref[...])
    clo[...] = y[:, :H]
    chi[...] = y[:, H:]


def _edgemm(eflo, efhi, W2, b2):
    return pl.pallas_call(
        _edgemm_body,
        grid=(E // BE,),
        in_specs=[pl.BlockSpec((BE, H), lambda i: (i, 0)),
                  pl.BlockSpec((BE, H), lambda i: (i, 0)),
                  pl.BlockSpec((D, D), lambda i: (0, 0)),
                  pl.BlockSpec((1, D), lambda i: (0, 0))],
        out_specs=[pl.BlockSpec((BE, H), lambda i: (i, 0)),
                   pl.BlockSpec((BE, H), lambda i: (i, 0))],
        out_shape=[jax.ShapeDtypeStruct((E, H), F32)] * 2,
    )(eflo, efhi, W2, b2.reshape(1, D))


def _edgebn_body(enlo, enhi, eflo, efhi, ssum, ssq, g_ref, bt_ref, olo, ohi):
    ss = ssum[...]
    qq = ssq[...]
    mlo = jnp.sum(ss[:NSUB], axis=0, keepdims=True) * (1.0 / E)
    mhi = jnp.sum(ss[NSUB:], axis=0, keepdims=True) * (1.0 / E)
    vlo = jnp.sum(qq[:NSUB], axis=0, keepdims=True) * (1.0 / E) - mlo * mlo
    vhi = jnp.sum(qq[NSUB:], axis=0, keepdims=True) * (1.0 / E) - mhi * mhi
    rlo = lax.rsqrt(vlo + 1e-5)
    rhi = lax.rsqrt(vhi + 1e-5)
    g = g_ref[...]
    bt = bt_ref[...]
    olo[...] = jnp.maximum((enlo[...] - mlo) * rlo * g[:, :H] + bt[:, :H],
                           0.0) + eflo[...]
    ohi[...] = jnp.maximum((enhi[...] - mhi) * rhi * g[:, H:] + bt[:, H:],
                           0.0) + efhi[...]


def _edgebn(enlo, enhi, eflo, efhi, ssum, ssq, g, bt):
    eb = pl.BlockSpec((BE, H), lambda i: (i, 0))
    return pl.pallas_call(
        _edgebn_body,
        grid=(E // BE,),
        in_specs=[eb, eb, eb, eb,
                  pl.BlockSpec((2 * NSUB, H), lambda i: (0, 0)),
                  pl.BlockSpec((2 * NSUB, H), lambda i: (0, 0)),
                  pl.BlockSpec((1, D), lambda i: (0, 0)),
                  pl.BlockSpec((1, D), lambda i: (0, 0))],
        out_specs=[eb, eb],
        out_shape=[jax.ShapeDtypeStruct((E, H), F32)] * 2,
    )(enlo, enhi, eflo, efhi, ssum, ssq, g.reshape(1, D), bt.reshape(1, D))


def _h_update(ah, nlo, nhi, dlo, dhi, hin, g, bt):
    num = jnp.concatenate([nlo, nhi], axis=1)
    den = jnp.concatenate([dlo, dhi], axis=1)
    x = ah + num / (den + 1e-6)
    mu = jnp.mean(x, axis=0, keepdims=True)
    var = jnp.mean(x * x, axis=0, keepdims=True) - mu * mu
    x = (x - mu) * lax.rsqrt(var + 1e-5) * g + bt
    return jnp.maximum(x, 0.0) + hin


def _hpost_body(ah, nlo, nhi, dlo, dhi, hin, g, bt, o_ref):
    o_ref[...] = _h_update(ah[...], nlo[...], nhi[...], dlo[...], dhi[...],
                           hin[...], g[...], bt[...])


def _hpost(ah, nlo, nhi, dlo, dhi, hin, g, bt):
    return pl.pallas_call(
        _hpost_body,
        out_shape=jax.ShapeDtypeStruct((N, D), F32),
    )(ah, nlo, nhi, dlo, dhi, hin, g.reshape(1, D), bt.reshape(1, D))


def _hlast_body(ah, nlo, nhi, dlo, dhi, hin, g, bt,
                w1, b1, w2, b2, w3, b3, o_ref):
    hf = _h_update(ah[...], nlo[...], nhi[...], dlo[...], dhi[...],
                   hin[...], g[...], bt[...])
    hg = jnp.mean(hf, axis=0, keepdims=True)
    t = jnp.maximum(jnp.dot(hg, w1[...], preferred_element_type=F32)
                    + b1[...], 0.0)
    t = jnp.maximum(jnp.dot(t, w2[...], preferred_element_type=F32)
                    + b2[...], 0.0)
    o_ref[...] = jnp.dot(t, w3[...], preferred_element_type=F32) + b3[...]


def _hlast(ah, nlo, nhi, dlo, dhi, hin, g, bt, W1, b1, W2, b2, W3, b3):
    return pl.pallas_call(
        _hlast_body,
        out_shape=jax.ShapeDtypeStruct((1, NC), F32),
    )(ah, nlo, nhi, dlo, dhi, hin, g.reshape(1, D), bt.reshape(1, D),
      W1, b1.reshape(1, D // 2), W2, b2.reshape(1, D // 4),
      W3, b3.reshape(1, NC))


def _sc_edge_body(src2, dst2, blo, bhi, dlo, dhi, elo, ehi, clo, chi,
                  enlo, enhi, nlo, nhi, delo, dehi, ssum, ssq,
                  sv0, sv1, dg0, dg1, ds0, ds1,
                  db0, db1, eb0, eb1, cb0, cb1, nb0, nb1, statv,
                  nacc, dacc,
                  ig0, ig1, is0, is1, gs0, gs1, ws0, ws1, ss0, ss1):
    c = lax.axis_index("c")
    s = lax.axis_index("s")
    z16 = jnp.zeros((16,), F32)
    SV = (sv0, sv1)
    DG = (dg0, dg1)
    DS = (ds0, ds1)
    IG = (ig0, ig1)
    IS = (is0, is1)
    DB = (db0, db1)
    EB = (eb0, eb1)
    CB = (cb0, cb1)
    NB = (nb0, nb1)
    GS = (gs0, gs1)
    WS = (ws0, ws1)
    SS = (ss0, ss1)

    for kk in range(4):
        statv[0, pl.ds(kk * 16, 16)] = z16
        statv[1, pl.ds(kk * 16, 16)] = z16

    def zrow(r, carry):
        for kk in range(4):
            cb0[r, pl.ds(kk * 16, 16)] = z16
        return carry

    lax.fori_loop(0, C, zrow, 0)
    for acc_ref in (nacc, dacc):
        done = 0
        while done < NPT:
            w = min(C, NPT - done)
            pltpu.sync_copy(cb0.at[pl.ds(0, w)],
                            acc_ref.at[pl.ds(s * NPT + done, w)])
            done += w
    plsc.subcore_barrier()

    def do_half(tb, td, te, tc, eno, nmo, dno, srow):
        def fire_idx_g(j, p):
            r = s * NCH + j
            pltpu.async_copy(src2.at[r], SV[p], IG[p])
            pltpu.async_copy(dst2.at[r], DG[p], IG[p])

        def wait_idx_g(p):
            pltpu.make_async_copy(src2.at[0], SV[p], IG[p]).wait()
            pltpu.make_async_copy(src2.at[0], DG[p], IG[p]).wait()

        def fire_idx_s(j, p):
            pltpu.async_copy(dst2.at[s * NCH + j], DS[p], IS[p])

        def wait_idx_s(p):
            pltpu.make_async_copy(src2.at[0], DS[p], IS[p]).wait()

        def fire_gathers(j, p):
            e0 = (s * NCH + j) * C
            pltpu.async_copy(tb.at[SV[p]], NB[p], GS[p])
            pltpu.async_copy(td.at[SV[p]], DB[p], GS[p])
            pltpu.async_copy(te.at[DG[p]], EB[p], GS[p])
            pltpu.async_copy(tc.at[pl.ds(e0, C)], CB[p], GS[p])

        def wait_gathers(p):
            for dref in (NB[p], DB[p], EB[p], CB[p]):
                pltpu.make_async_copy(tc.at[pl.ds(0, C)], dref, GS[p]).wait()

        def fire_out(j, p):
            e0 = (s * NCH + j) * C
            pltpu.async_copy(CB[p], eno.at[pl.ds(e0, C)], WS[p])
            pltpu.async_copy(NB[p], nacc.at[DS[p]], SS[p], add=True)
            pltpu.async_copy(EB[p], dacc.at[DS[p]], SS[p], add=True)

        def wait_out(p):
            pltpu.make_async_copy(CB[p], eno.at[pl.ds(0, C)], WS[p]).wait()
            pltpu.make_async_copy(NB[p], nmo.at[pl.ds(0, C)], SS[p]).wait()
            pltpu.make_async_copy(EB[p], dno.at[pl.ds(0, C)], SS[p]).wait()

        def compute(p):
            def rowfn(rr, acc):
                out = []
                for kk in range(4):
                    sl = pl.ds(kk * 16, 16)
                    x = CB[p][rr, sl] + DB[p][rr, sl] + EB[p][rr, sl]
                    CB[p][rr, sl] = x
                    sg = 1.0 / (1.0 + jnp.exp(-x))
                    EB[p][rr, sl] = sg
                    NB[p][rr, sl] = sg * NB[p][rr, sl]
                    out.append((acc[kk] + x, acc[kk + 4] + x * x))
                return tuple(a for a, _ in out) + tuple(b for _, b in out)

            acc = lax.fori_loop(0, C, rowfn,
                                tuple(jnp.zeros((16,), F32) for _ in range(8)))
            for kk in range(4):
                sl = pl.ds(kk * 16, 16)
                statv[0, sl] += acc[kk]
                statv[1, sl] += acc[kk + 4]

        fire_idx_g(0, 0)
        wait_idx_g(0)
        fire_gathers(0, 0)
        fire_idx_g(1, 1)
        fire_idx_s(0, 0)

        def body2(t, carry):
            for p in (0, 1):
                j = t * 2 + p
                q = 1 - p
                wait_gathers(p)
                compute(p)
                wait_idx_s(p)
                fire_out(j, p)

                @pl.when(j + 1 < NCH)
                def _():
                    @pl.when(j >= 1)
                    def _():
                        wait_out(q)
                    fire_idx_s(j + 1, q)
                    wait_idx_g(q)
                    fire_gathers(j + 1, q)

                    @pl.when(j + 2 < NCH)
                    def _():
                        fire_idx_g(j + 2, p)
            return carry

        lax.fori_loop(0, NCH // 2, body2, 0)
        wait_out(0)
        wait_out(1)
        plsc.subcore_barrier()
        done = 0
        while done < NPT:
            w = min(C, NPT - done)
            sl = pl.ds(s * NPT + done, w)
            pltpu.sync_copy(nacc.at[sl], nmo.at[sl])
            pltpu.sync_copy(dacc.at[sl], dno.at[sl])
            done += w
        pltpu.sync_copy(statv.at[pl.ds(0, 1)], ssum.at[pl.ds(srow + s, 1)])
        pltpu.sync_copy(statv.at[pl.ds(1, 1)], ssq.at[pl.ds(srow + s, 1)])

    @pl.when(c == 0)
    def _():
        do_half(blo, dlo, elo, clo, enlo, nlo, delo, 0)

    @pl.when(c == 1)
    def _():
        do_half(bhi, dhi, ehi, chi, enhi, nhi, dehi, NSUB)


def _sc_edge(src2, dst2, blo, bhi, dlo, dhi, elo, ehi, clo, chi):
    mesh = plsc.VectorSubcoreMesh(core_axis_name="c", subcore_axis_name="s",
                                  num_cores=2, num_subcores=NSUB)
    out_type = [jax.ShapeDtypeStruct((E, H), F32),
                jax.ShapeDtypeStruct((E, H), F32),
                jax.ShapeDtypeStruct((N, H), F32),
                jax.ShapeDtypeStruct((N, H), F32),
                jax.ShapeDtypeStruct((N, H), F32),
                jax.ShapeDtypeStruct((N, H), F32),
                jax.ShapeDtypeStruct((2 * NSUB, H), F32),
                jax.ShapeDtypeStruct((2 * NSUB, H), F32)]
    scratch = ([pltpu.VMEM((C,), jnp.int32)] * 6
               + [pltpu.VMEM((C, H), F32)] * 8
               + [pltpu.VMEM((2, H), F32),
                  pltpu.VMEM_SHARED((N, H), F32),
                  pltpu.VMEM_SHARED((N, H), F32)]
               + [pltpu.SemaphoreType.DMA] * 10)
    fn = pl.kernel(_sc_edge_body, out_type=out_type, mesh=mesh,
                   scratch_types=scratch,
                   compiler_params=pltpu.CompilerParams(
                       use_tc_tiling_on_sc=False))
    return fn(src2, dst2, blo, bhi, dlo, dhi, elo, ehi, clo, chi)


def kernel(h, e, edge_index, Wh, bh, We, be, Wl, bl, gamma, beta,
           W1, b1, W2, b2, W3, b3):
    src2 = edge_index[0].reshape(E // C, C)
    dst2 = edge_index[1].reshape(E // C, C)
    hcur = _pe(h, Wh, bh)
    row0 = We[0] + be
    eflo = jnp.broadcast_to(row0[:H], (E, H))
    efhi = jnp.broadcast_to(row0[H:], (E, H))
    logits = None
    for l in range(L):
        ah, blo, bhi, dlo, dhi, elo, ehi = _node_mm(hcur, Wl[l], bl[l])
        clo, chi = _edgemm(eflo, efhi, Wl[l, 2], bl[l, 2])
        enlo, enhi, nlo, nhi, delo, dehi, ssum, ssq = _sc_edge(
            src2, dst2, blo, bhi, dlo, dhi, elo, ehi, clo, chi)
        if l < L - 1:
            hcur = _hpost(ah, nlo, nhi, delo, dehi, hcur,
                          gamma[l, 0], beta[l, 0])
            eflo, efhi = _edgebn(enlo, enhi, eflo, efhi, ssum, ssq,
                                 gamma[l, 1], beta[l, 1])
        else:
            logits = _hlast(ah, nlo, nhi, delo, dehi, hcur,
                            gamma[l, 0], beta[l, 0],
                            W1, b1, W2, b2, W3, b3)
    return logits.reshape(NC)


# speedup vs baseline: 2.7829x; 1.2508x over previous
import functools

import jax
import jax.numpy as jnp
from jax import lax
from jax.experimental import pallas as pl
from jax.experimental.pallas import tpu as pltpu
from jax.experimental.pallas import tpu_sc as plsc

N = 10000
E = 320000
D = 128
H = 64
L = 4
NC = 10
C = 80
NSUB = 16
EPT = E // NSUB
NCH = EPT // C
NPT = N // NSUB
BE = 3200
F32 = jnp.float32


def _pe_body(h_ref, w_ref, b_ref, o_ref):
    o_ref[...] = jnp.dot(h_ref[...], w_ref[...],
                         preferred_element_type=F32) + b_ref[...]


def _pe(h, Wh, bh):
    return pl.pallas_call(
        _pe_body,
        out_shape=jax.ShapeDtypeStruct((N, D), F32),
    )(h, Wh, bh.reshape(1, D))


def _node_mm_body(h_ref, w0, w1, w3, w4, b0, b1, b3, b4,
                  ah_ref, blo, bhi, dlo, dhi, elo, ehi):
    x = h_ref[...]
    A = jnp.dot(x, w0[...], preferred_element_type=F32) + b0[...]
    B = jnp.dot(x, w1[...], preferred_element_type=F32) + b1[...]
    Dm = jnp.dot(x, w3[...], preferred_element_type=F32) + b3[...]
    Em = jnp.dot(x, w4[...], preferred_element_type=F32) + b4[...]
    ah_ref[...] = A
    blo[...] = B[:, :H]
    bhi[...] = B[:, H:]
    dlo[...] = Dm[:, :H]
    dhi[...] = Dm[:, H:]
    elo[...] = Em[:, :H]
    ehi[...] = Em[:, H:]


def _node_mm(h, Wl_l, bl_l):
    outs = [jax.ShapeDtypeStruct((N, D), F32)] + \
           [jax.ShapeDtypeStruct((N, H), F32)] * 6
    return pl.pallas_call(
        _node_mm_body,
        out_shape=outs,
    )(h, Wl_l[0], Wl_l[1], Wl_l[3], Wl_l[4],
      bl_l[0].reshape(1, D), bl_l[1].reshape(1, D),
      bl_l[3].reshape(1, D), bl_l[4].reshape(1, D))


def _edgemm_body(eflo, efhi, w_ref, b_ref, clo, chi):
    w = w_ref[...]
    y = (jnp.dot(eflo[...], w[:H], preferred_element_type=F32)
         + jnp.dot(efhi[...], w[H:], preferred_element_type=F32)
         + b_ref[...])
    clo[...] = y[:, :H]
    chi[...] = y[:, H:]


def _edgemm(eflo, efhi, W2, b2):
    return pl.pallas_call(
        _edgemm_body,
        grid=(E // BE,),
        in_specs=[pl.BlockSpec((BE, H), lambda i: (i, 0)),
                  pl.BlockSpec((BE, H), lambda i: (i, 0)),
                  pl.BlockSpec((D, D), lambda i: (0, 0)),
                  pl.BlockSpec((1, D), lambda i: (0, 0))],
        out_specs=[pl.BlockSpec((BE, H), lambda i: (i, 0)),
                   pl.BlockSpec((BE, H), lambda i: (i, 0))],
        out_shape=[jax.ShapeDtypeStruct((E, H), F32)] * 2,
    )(eflo, efhi, W2, b2.reshape(1, D))


def _edgebn_body(enlo, enhi, eflo, efhi, ssum, ssq, g_ref, bt_ref, olo, ohi):
    ss = ssum[...]
    qq = ssq[...]
    mlo = jnp.sum(ss[:NSUB], axis=0, keepdims=True) * (1.0 / E)
    mhi = jnp.sum(ss[NSUB:], axis=0, keepdims=True) * (1.0 / E)
    vlo = jnp.sum(qq[:NSUB], axis=0, keepdims=True) * (1.0 / E) - mlo * mlo
    vhi = jnp.sum(qq[NSUB:], axis=0, keepdims=True) * (1.0 / E) - mhi * mhi
    rlo = lax.rsqrt(vlo + 1e-5)
    rhi = lax.rsqrt(vhi + 1e-5)
    g = g_ref[...]
    bt = bt_ref[...]
    olo[...] = jnp.maximum((enlo[...] - mlo) * rlo * g[:, :H] + bt[:, :H],
                           0.0) + eflo[...]
    ohi[...] = jnp.maximum((enhi[...] - mhi) * rhi * g[:, H:] + bt[:, H:],
                           0.0) + efhi[...]


def _edgebn(enlo, enhi, eflo, efhi, ssum, ssq, g, bt):
    eb = pl.BlockSpec((BE, H), lambda i: (i, 0))
    return pl.pallas_call(
        _edgebn_body,
        grid=(E // BE,),
        in_specs=[eb, eb, eb, eb,
                  pl.BlockSpec((2 * NSUB, H), lambda i: (0, 0)),
                  pl.BlockSpec((2 * NSUB, H), lambda i: (0, 0)),
                  pl.BlockSpec((1, D), lambda i: (0, 0)),
                  pl.BlockSpec((1, D), lambda i: (0, 0))],
        out_specs=[eb, eb],
        out_shape=[jax.ShapeDtypeStruct((E, H), F32)] * 2,
    )(enlo, enhi, eflo, efhi, ssum, ssq, g.reshape(1, D), bt.reshape(1, D))


def _edgebnmm_body(enlo, enhi, eflo, efhi, ssum, ssq, g_ref, bt_ref,
                   w_ref, b_ref, olo, ohi, clo, chi):
    ss = ssum[...]
    qq = ssq[...]
    mlo = jnp.sum(ss[:NSUB], axis=0, keepdims=True) * (1.0 / E)
    mhi = jnp.sum(ss[NSUB:], axis=0, keepdims=True) * (1.0 / E)
    vlo = jnp.sum(qq[:NSUB], axis=0, keepdims=True) * (1.0 / E) - mlo * mlo
    vhi = jnp.sum(qq[NSUB:], axis=0, keepdims=True) * (1.0 / E) - mhi * mhi
    rlo = lax.rsqrt(vlo + 1e-5)
    rhi = lax.rsqrt(vhi + 1e-5)
    g = g_ref[...]
    bt = bt_ref[...]
    xlo = jnp.maximum((enlo[...] - mlo) * rlo * g[:, :H] + bt[:, :H],
                      0.0) + eflo[...]
    xhi = jnp.maximum((enhi[...] - mhi) * rhi * g[:, H:] + bt[:, H:],
                      0.0) + efhi[...]
    olo[...] = xlo
    ohi[...] = xhi
    w = w_ref[...]
    y = (jnp.dot(xlo, w[:H], preferred_element_type=F32)
         + jnp.dot(xhi, w[H:], preferred_element_type=F32) + b_ref[...])
    clo[...] = y[:, :H]
    chi[...] = y[:, H:]


def _edgebnmm(enlo, enhi, eflo, efhi, ssum, ssq, g, bt, W2, b2):
    eb = pl.BlockSpec((BE, H), lambda i: (i, 0))
    return pl.pallas_call(
        _edgebnmm_body,
        grid=(E // BE,),
        in_specs=[eb, eb, eb, eb,
                  pl.BlockSpec((2 * NSUB, H), lambda i: (0, 0)),
                  pl.BlockSpec((2 * NSUB, H), lambda i: (0, 0)),
                  pl.BlockSpec((1, D), lambda i: (0, 0)),
                  pl.BlockSpec((1, D), lambda i: (0, 0)),
                  pl.BlockSpec((D, D), lambda i: (0, 0)),
                  pl.BlockSpec((1, D), lambda i: (0, 0))],
        out_specs=[eb, eb, eb, eb],
        out_shape=[jax.ShapeDtypeStruct((E, H), F32)] * 4,
    )(enlo, enhi, eflo, efhi, ssum, ssq, g.reshape(1, D), bt.reshape(1, D),
      W2, b2.reshape(1, D))


def _h_update(ah, nlo, nhi, dlo, dhi, hin, g, bt):
    num = jnp.concatenate([nlo, nhi], axis=1)
    den = jnp.concatenate([dlo, dhi], axis=1)
    x = ah + num / (den + 1e-6)
    mu = jnp.mean(x, axis=0, keepdims=True)
    var = jnp.mean(x * x, axis=0, keepdims=True) - mu * mu
    x = (x - mu) * lax.rsqrt(var + 1e-5) * g + bt
    return jnp.maximum(x, 0.0) + hin


def _hpost_body(ah, nlo, nhi, dlo, dhi, hin, g, bt, o_ref):
    o_ref[...] = _h_update(ah[...], nlo[...], nhi[...], dlo[...], dhi[...],
                           hin[...], g[...], bt[...])


def _hpost(ah, nlo, nhi, dlo, dhi, hin, g, bt):
    return pl.pallas_call(
        _hpost_body,
        out_shape=jax.ShapeDtypeStruct((N, D), F32),
    )(ah, nlo, nhi, dlo, dhi, hin, g.reshape(1, D), bt.reshape(1, D))


def _hlast_body(ah, nlo, nhi, dlo, dhi, hin, g, bt,
                w1, b1, w2, b2, w3, b3, o_ref):
    hf = _h_update(ah[...], nlo[...], nhi[...], dlo[...], dhi[...],
                   hin[...], g[...], bt[...])
    hg = jnp.mean(hf, axis=0, keepdims=True)
    t = jnp.maximum(jnp.dot(hg, w1[...], preferred_element_type=F32)
                    + b1[...], 0.0)
    t = jnp.maximum(jnp.dot(t, w2[...], preferred_element_type=F32)
                    + b2[...], 0.0)
    o_ref[...] = jnp.dot(t, w3[...], preferred_element_type=F32) + b3[...]


def _hlast(ah, nlo, nhi, dlo, dhi, hin, g, bt, W1, b1, W2, b2, W3, b3):
    return pl.pallas_call(
        _hlast_body,
        out_shape=jax.ShapeDtypeStruct((1, NC), F32),
    )(ah, nlo, nhi, dlo, dhi, hin, g.reshape(1, D), bt.reshape(1, D),
      W1, b1.reshape(1, D // 2), W2, b2.reshape(1, D // 4),
      W3, b3.reshape(1, NC))


def _sc_edge_body(src2, dst2, blo, bhi, dlo, dhi, elo, ehi, clo, chi,
                  enlo, enhi, nlo, nhi, delo, dehi, ssum, ssq,
                  sv0, sv1, dg0, dg1, ds0, ds1,
                  db0, db1, eb0, eb1, cb0, cb1, nb0, nb1, statv,
                  nacc, dacc,
                  ig0, ig1, is0, is1, gs0, gs1, ws0, ws1, ss0, ss1):
    c = lax.axis_index("c")
    s = lax.axis_index("s")
    z16 = jnp.zeros((16,), F32)
    SV = (sv0, sv1)
    DG = (dg0, dg1)
    DS = (ds0, ds1)
    IG = (ig0, ig1)
    IS = (is0, is1)
    DB = (db0, db1)
    EB = (eb0, eb1)
    CB = (cb0, cb1)
    NB = (nb0, nb1)
    GS = (gs0, gs1)
    WS = (ws0, ws1)
    SS = (ss0, ss1)

    for kk in range(4):
        statv[0, pl.ds(kk * 16, 16)] = z16
        statv[1, pl.ds(kk * 16, 16)] = z16

    def zrow(r, carry):
        for kk in range(4):
            cb0[r, pl.ds(kk * 16, 16)] = z16
        return carry

    lax.fori_loop(0, C, zrow, 0)
    for acc_ref in (nacc, dacc):
        done = 0
        while done < NPT:
            w = min(C, NPT - done)
            pltpu.sync_copy(cb0.at[pl.ds(0, w)],
                            acc_ref.at[pl.ds(s * NPT + done, w)])
            done += w
    plsc.subcore_barrier()

    def do_half(tb, td, te, tc, eno, nmo, dno, srow):
        def fire_idx_g(j, p):
            r = s * NCH + j
            pltpu.async_copy(src2.at[r], SV[p], IG[p])
            pltpu.async_copy(dst2.at[r], DG[p], IG[p])

        def wait_idx_g(p):
            pltpu.make_async_copy(src2.at[0], SV[p], IG[p]).wait()
            pltpu.make_async_copy(src2.at[0], DG[p], IG[p]).wait()

        def fire_idx_s(j, p):
            pltpu.async_copy(dst2.at[s * NCH + j], DS[p], IS[p])

        def wait_idx_s(p):
            pltpu.make_async_copy(src2.at[0], DS[p], IS[p]).wait()

        def fire_gathers(j, p):
            e0 = (s * NCH + j) * C
            pltpu.async_copy(tb.at[SV[p]], NB[p], GS[p])
            pltpu.async_copy(td.at[SV[p]], DB[p], GS[p])
            pltpu.async_copy(te.at[DG[p]], EB[p], GS[p])
            pltpu.async_copy(tc.at[pl.ds(e0, C)], CB[p], GS[p])

        def wait_gathers(p):
            for dref in (NB[p], DB[p], EB[p], CB[p]):
                pltpu.make_async_copy(tc.at[pl.ds(0, C)], dref, GS[p]).wait()

        def fire_out(j, p):
            e0 = (s * NCH + j) * C
            pltpu.async_copy(CB[p], eno.at[pl.ds(e0, C)], WS[p])
            pltpu.async_copy(NB[p], nacc.at[DS[p]], SS[p], add=True)
            pltpu.async_copy(EB[p], dacc.at[DS[p]], SS[p], add=True)

        def wait_out(p):
            pltpu.make_async_copy(CB[p], eno.at[pl.ds(0, C)], WS[p]).wait()
            pltpu.make_async_copy(NB[p], nmo.at[pl.ds(0, C)], SS[p]).wait()
            pltpu.make_async_copy(EB[p], dno.at[pl.ds(0, C)], SS[p]).wait()

        def compute(p):
            def rowfn(rr, acc):
                out = []
                for kk in range(4):
                    sl = pl.ds(kk * 16, 16)
                    x = CB[p][rr, sl] + DB[p][rr, sl] + EB[p][rr, sl]
                    CB[p][rr, sl] = x
                    sg = 1.0 / (1.0 + jnp.exp(-x))
                    EB[p][rr, sl] = sg
                    NB[p][rr, sl] = sg * NB[p][rr, sl]
                    out.append((acc[kk] + x, acc[kk + 4] + x * x))
                return tuple(a for a, _ in out) + tuple(b for _, b in out)

            acc = lax.fori_loop(0, C, rowfn,
                                tuple(jnp.zeros((16,), F32) for _ in range(8)))
            for kk in range(4):
                sl = pl.ds(kk * 16, 16)
                statv[0, sl] += acc[kk]
                statv[1, sl] += acc[kk + 4]

        fire_idx_g(0, 0)
        wait_idx_g(0)
        fire_gathers(0, 0)
        fire_idx_g(1, 1)
        fire_idx_s(0, 0)

        def body2(t, carry):
            for p in (0, 1):
                j = t * 2 + p
                q = 1 - p
                wait_gathers(p)

                @pl.when(j + 1 < NCH)
                def _():
                    @pl.when(j >= 1)
                    def _():
                        wait_out(q)
                    fire_idx_s(j + 1, q)
                    wait_idx_g(q)
                    fire_gathers(j + 1, q)

                    @pl.when(j + 2 < NCH)
                    def _():
                        fire_idx_g(j + 2, p)

                compute(p)
                wait_idx_s(p)
                fire_out(j, p)
            return carry

        lax.fori_loop(0, NCH // 2, body2, 0)
        wait_out(0)
        wait_out(1)
        plsc.subcore_barrier()
        done = 0
        while done < NPT:
            w = min(C, NPT - done)
            sl = pl.ds(s * NPT + done, w)
            pltpu.sync_copy(nacc.at[sl], nmo.at[sl])
            pltpu.sync_copy(dacc.at[sl], dno.at[sl])
            done += w
        pltpu.sync_copy(statv.at[pl.ds(0, 1)], ssum.at[pl.ds(srow + s, 1)])
        pltpu.sync_copy(statv.at[pl.ds(1, 1)], ssq.at[pl.ds(srow + s, 1)])

    @pl.when(c == 0)
    def _():
        do_half(blo, dlo, elo, clo, enlo, nlo, delo, 0)

    @pl.when(c == 1)
    def _():
        do_half(bhi, dhi, ehi, chi, enhi, nhi, dehi, NSUB)


def _sc_edge(src2, dst2, blo, bhi, dlo, dhi, elo, ehi, clo, chi):
    mesh = plsc.VectorSubcoreMesh(core_axis_name="c", subcore_axis_name="s",
                                  num_cores=2, num_subcores=NSUB)
    out_type = [jax.ShapeDtypeStruct((E, H), F32),
                jax.ShapeDtypeStruct((E, H), F32),
                jax.ShapeDtypeStruct((N, H), F32),
                jax.ShapeDtypeStruct((N, H), F32),
                jax.ShapeDtypeStruct((N, H), F32),
                jax.ShapeDtypeStruct((N, H), F32),
                jax.ShapeDtypeStruct((2 * NSUB, H), F32),
                jax.ShapeDtypeStruct((2 * NSUB, H), F32)]
    scratch = ([pltpu.VMEM((C,), jnp.int32)] * 6
               + [pltpu.VMEM((C, H), F32)] * 8
               + [pltpu.VMEM((2, H), F32),
                  pltpu.VMEM_SHARED((N, H), F32),
                  pltpu.VMEM_SHARED((N, H), F32)]
               + [pltpu.SemaphoreType.DMA] * 10)
    fn = pl.kernel(_sc_edge_body, out_type=out_type, mesh=mesh,
                   scratch_types=scratch,
                   compiler_params=pltpu.CompilerParams(
                       use_tc_tiling_on_sc=False))
    return fn(src2, dst2, blo, bhi, dlo, dhi, elo, ehi, clo, chi)


def kernel(h, e, edge_index, Wh, bh, We, be, Wl, bl, gamma, beta,
           W1, b1, W2, b2, W3, b3):
    src2 = edge_index[0].reshape(E // C, C)
    dst2 = edge_index[1].reshape(E // C, C)
    hcur = _pe(h, Wh, bh)
    row0 = We[0] + be
    eflo = jnp.broadcast_to(row0[:H], (E, H))
    efhi = jnp.broadcast_to(row0[H:], (E, H))
    logits = None
    clo, chi = _edgemm(eflo, efhi, Wl[0, 2], bl[0, 2])
    for l in range(L):
        ah, blo, bhi, dlo, dhi, elo, ehi = _node_mm(hcur, Wl[l], bl[l])
        enlo, enhi, nlo, nhi, delo, dehi, ssum, ssq = _sc_edge(
            src2, dst2, blo, bhi, dlo, dhi, elo, ehi, clo, chi)
        if l < L - 1:
            hcur = _hpost(ah, nlo, nhi, delo, dehi, hcur,
                          gamma[l, 0], beta[l, 0])
            eflo, efhi, clo, chi = _edgebnmm(
                enlo, enhi, eflo, efhi, ssum, ssq,
                gamma[l, 1], beta[l, 1], Wl[l + 1, 2], bl[l + 1, 2])
        else:
            logits = _hlast(ah, nlo, nhi, delo, dehi, hcur,
                            gamma[l, 0], beta[l, 0],
                            W1, b1, W2, b2, W3, b3)
    return logits.reshape(NC)


# speedup vs baseline: 5.4133x; 1.9452x over previous
import functools

import jax
import jax.numpy as jnp
from jax import lax
from jax.experimental import pallas as pl
from jax.experimental.pallas import tpu as pltpu
from jax.experimental.pallas import tpu_sc as plsc

N = 10000
E = 320000
D = 128
H = 64
L = 4
NC = 10
C = 80
NSUB = 16
EPT = E // NSUB
NCH = EPT // C
NPT = N // NSUB
BE = 1600
F32 = jnp.float32


def _pe_body(h_ref, w_ref, b_ref, o_ref):
    o_ref[...] = jnp.dot(h_ref[...], w_ref[...],
                         preferred_element_type=F32) + b_ref[...]


def _pe(h, Wh, bh):
    return pl.pallas_call(
        _pe_body,
        out_shape=jax.ShapeDtypeStruct((N, D), F32),
    )(h, Wh, bh.reshape(1, D))


def _node_mm_body(h_ref, w0, w1, w3, w4, b0, b1, b3, b4,
                  ah_ref, blo, bhi, dlo, dhi, elo, ehi):
    x = h_ref[...]
    A = jnp.dot(x, w0[...], preferred_element_type=F32) + b0[...]
    B = jnp.dot(x, w1[...], preferred_element_type=F32) + b1[...]
    Dm = jnp.dot(x, w3[...], preferred_element_type=F32) + b3[...]
    Em = jnp.dot(x, w4[...], preferred_element_type=F32) + b4[...]
    ah_ref[...] = A
    blo[...] = B[:, :H]
    bhi[...] = B[:, H:]
    dlo[...] = Dm[:, :H]
    dhi[...] = Dm[:, H:]
    elo[...] = Em[:, :H]
    ehi[...] = Em[:, H:]


def _node_mm(h, Wl_l, bl_l):
    outs = [jax.ShapeDtypeStruct((N, D), F32)] + \
           [jax.ShapeDtypeStruct((N, H), F32)] * 6
    return pl.pallas_call(
        _node_mm_body,
        out_shape=outs,
    )(h, Wl_l[0], Wl_l[1], Wl_l[3], Wl_l[4],
      bl_l[0].reshape(1, D), bl_l[1].reshape(1, D),
      bl_l[3].reshape(1, D), bl_l[4].reshape(1, D))


def _bd_weights(W2, b2):
    eye2 = jnp.eye(2, dtype=F32)
    wa = jnp.kron(eye2, W2[:H, :H])
    wb = jnp.kron(eye2, W2[H:, :H])
    wc = jnp.kron(eye2, W2[:H, H:])
    wd = jnp.kron(eye2, W2[H:, H:])
    blo = jnp.tile(b2[:H], 2).reshape(1, D)
    bhi = jnp.tile(b2[H:], 2).reshape(1, D)
    return wa, wb, wc, wd, blo, bhi


def _edgemm_body(eflo, efhi, wa, wb, wc, wd, blo_ref, bhi_ref, clo, chi):
    xlo = eflo[...]
    xhi = efhi[...]
    clo[...] = (jnp.dot(xlo, wa[...], preferred_element_type=F32)
                + jnp.dot(xhi, wb[...], preferred_element_type=F32)
                + blo_ref[...])
    chi[...] = (jnp.dot(xlo, wc[...], preferred_element_type=F32)
                + jnp.dot(xhi, wd[...], preferred_element_type=F32)
                + bhi_ref[...])


def _edgemm(eflo, efhi, W2, b2):
    eb = pl.BlockSpec((BE, D), lambda i: (i, 0))
    wf = pl.BlockSpec((D, D), lambda i: (0, 0))
    rf = pl.BlockSpec((1, D), lambda i: (0, 0))
    wa, wb, wc, wd, blo, bhi = _bd_weights(W2, b2)
    return pl.pallas_call(
        _edgemm_body,
        grid=(E // 2 // BE,),
        in_specs=[eb, eb, wf, wf, wf, wf, rf, rf],
        out_specs=[eb, eb],
        out_shape=[jax.ShapeDtypeStruct((E // 2, D), F32)] * 2,
    )(eflo, efhi, wa, wb, wc, wd, blo, bhi)


def _edgebnmm_body(enlo, enhi, eflo, efhi, ssum, ssq, g2lo, g2hi,
                   b2lo, b2hi, wa, wb, wc, wd, mblo, mbhi,
                   olo, ohi, clo, chi):
    ss = ssum[:, 0, :]
    qq = ssq[:, 0, :]
    mlo = jnp.sum(ss[:NSUB], axis=0, keepdims=True) * (1.0 / E)
    mhi = jnp.sum(ss[NSUB:], axis=0, keepdims=True) * (1.0 / E)
    vlo = jnp.sum(qq[:NSUB], axis=0, keepdims=True) * (1.0 / E) - mlo * mlo
    vhi = jnp.sum(qq[NSUB:], axis=0, keepdims=True) * (1.0 / E) - mhi * mhi
    m2lo = jnp.concatenate([mlo, mlo], axis=1)
    m2hi = jnp.concatenate([mhi, mhi], axis=1)
    r2lo = lax.rsqrt(jnp.concatenate([vlo, vlo], axis=1) + 1e-5)
    r2hi = lax.rsqrt(jnp.concatenate([vhi, vhi], axis=1) + 1e-5)
    xlo = jnp.maximum((enlo[...] - m2lo) * r2lo * g2lo[...] + b2lo[...],
                      0.0) + eflo[...]
    xhi = jnp.maximum((enhi[...] - m2hi) * r2hi * g2hi[...] + b2hi[...],
                      0.0) + efhi[...]
    olo[...] = xlo
    ohi[...] = xhi
    clo[...] = (jnp.dot(xlo, wa[...], preferred_element_type=F32)
                + jnp.dot(xhi, wb[...], preferred_element_type=F32)
                + mblo[...])
    chi[...] = (jnp.dot(xlo, wc[...], preferred_element_type=F32)
                + jnp.dot(xhi, wd[...], preferred_element_type=F32)
                + mbhi[...])


def _edgebnmm(enlo, enhi, eflo, efhi, ssum, ssq, g, bt, W2, b2):
    eb = pl.BlockSpec((BE, D), lambda i: (i, 0))
    sf = pl.BlockSpec((2 * NSUB, 8, H), lambda i: (0, 0, 0))
    wf = pl.BlockSpec((D, D), lambda i: (0, 0))
    rf = pl.BlockSpec((1, D), lambda i: (0, 0))
    wa, wb, wc, wd, mblo, mbhi = _bd_weights(W2, b2)
    g2lo = jnp.tile(g[:H], 2).reshape(1, D)
    g2hi = jnp.tile(g[H:], 2).reshape(1, D)
    b2lo = jnp.tile(bt[:H], 2).reshape(1, D)
    b2hi = jnp.tile(bt[H:], 2).reshape(1, D)
    return pl.pallas_call(
        _edgebnmm_body,
        grid=(E // 2 // BE,),
        in_specs=[eb, eb, eb, eb, sf, sf, rf, rf, rf, rf,
                  wf, wf, wf, wf, rf, rf],
        out_specs=[eb, eb, eb, eb],
        out_shape=[jax.ShapeDtypeStruct((E // 2, D), F32)] * 4,
    )(enlo, enhi, eflo, efhi, ssum, ssq, g2lo, g2hi, b2lo, b2hi,
      wa, wb, wc, wd, mblo, mbhi)


def _h_update(ah, nlo, nhi, dlo, dhi, hin, g, bt):
    num = jnp.concatenate([nlo, nhi], axis=1)
    den = jnp.concatenate([dlo, dhi], axis=1)
    x = ah + num / (den + 1e-6)
    mu = jnp.mean(x, axis=0, keepdims=True)
    var = jnp.mean(x * x, axis=0, keepdims=True) - mu * mu
    x = (x - mu) * lax.rsqrt(var + 1e-5) * g + bt
    return jnp.maximum(x, 0.0) + hin


def _hpost_body(ah, nlo, nhi, dlo, dhi, hin, g, bt, o_ref):
    o_ref[...] = _h_update(ah[...], nlo[...], nhi[...], dlo[...], dhi[...],
                           hin[...], g[...], bt[...])


def _hpost(ah, nlo, nhi, dlo, dhi, hin, g, bt):
    return pl.pallas_call(
        _hpost_body,
        out_shape=jax.ShapeDtypeStruct((N, D), F32),
    )(ah, nlo, nhi, dlo, dhi, hin, g.reshape(1, D), bt.reshape(1, D))


def _hlast_body(ah, nlo, nhi, dlo, dhi, hin, g, bt,
                w1, b1, w2, b2, w3, b3, o_ref):
    hf = _h_update(ah[...], nlo[...], nhi[...], dlo[...], dhi[...],
                   hin[...], g[...], bt[...])
    hg = jnp.mean(hf, axis=0, keepdims=True)
    t = jnp.maximum(jnp.dot(hg, w1[...], preferred_element_type=F32)
                    + b1[...], 0.0)
    t = jnp.maximum(jnp.dot(t, w2[...], preferred_element_type=F32)
                    + b2[...], 0.0)
    o_ref[...] = jnp.dot(t, w3[...], preferred_element_type=F32) + b3[...]


def _hlast(ah, nlo, nhi, dlo, dhi, hin, g, bt, W1, b1, W2, b2, W3, b3):
    return pl.pallas_call(
        _hlast_body,
        out_shape=jax.ShapeDtypeStruct((1, NC), F32),
    )(ah, nlo, nhi, dlo, dhi, hin, g.reshape(1, D), bt.reshape(1, D),
      W1, b1.reshape(1, D // 2), W2, b2.reshape(1, D // 4),
      W3, b3.reshape(1, NC))


def _sc_edge_body(src2, dst2, blo, bhi, dlo, dhi, elo, ehi, clo, chi,
                  enlo, enhi, nlo, nhi, delo, dehi, ssum, ssq,
                  sv0, sv1, dg0, dg1, ds0, ds1,
                  db0, db1, eb0, eb1, cb0, cb1, nb0, nb1, statv,
                  nacc, dacc,
                  ig0, ig1, is0, is1, gs0, gs1, ws0, ws1, ss0, ss1):
    c = lax.axis_index("c")
    s = lax.axis_index("s")
    z16 = jnp.zeros((16,), F32)
    SV = (sv0, sv1)
    DG = (dg0, dg1)
    DS = (ds0, ds1)
    IG = (ig0, ig1)
    IS = (is0, is1)
    DB = (db0, db1)
    EB = (eb0, eb1)
    CB = (cb0, cb1)
    NB = (nb0, nb1)
    GS = (gs0, gs1)
    WS = (ws0, ws1)
    SS = (ss0, ss1)

    for kk in range(4):
        statv[0, pl.ds(kk * 16, 16)] = z16
        statv[1, pl.ds(kk * 16, 16)] = z16

    def zrow(r, carry):
        for kk in range(4):
            nb0[r, pl.ds(kk * 16, 16)] = z16
        return carry

    lax.fori_loop(0, C, zrow, 0)
    for acc_ref in (nacc, dacc):
        done = 0
        while done < NPT:
            w = min(C, NPT - done)
            pltpu.sync_copy(nb0.at[pl.ds(0, w)],
                            acc_ref.at[pl.ds(s * NPT + done, w)])
            done += w
    plsc.subcore_barrier()

    def do_half(tb, td, te, tc, eno, nmo, dno, srow):
        def fire_idx_g(j, p):
            r = s * NCH + j
            pltpu.async_copy(src2.at[r], SV[p], IG[p])
            pltpu.async_copy(dst2.at[r], DG[p], IG[p])

        def wait_idx_g(p):
            pltpu.make_async_copy(src2.at[0], SV[p], IG[p]).wait()
            pltpu.make_async_copy(src2.at[0], DG[p], IG[p]).wait()

        def fire_idx_s(j, p):
            pltpu.async_copy(dst2.at[s * NCH + j], DS[p], IS[p])

        def wait_idx_s(p):
            pltpu.make_async_copy(src2.at[0], DS[p], IS[p]).wait()

        def fire_gathers(j, p):
            r40 = (s * NCH + j) * (C // 2)
            pltpu.async_copy(tb.at[SV[p].at[0]], NB[p], GS[p])
            pltpu.async_copy(td.at[SV[p].at[0]], DB[p], GS[p])
            pltpu.async_copy(te.at[DG[p].at[0]], EB[p], GS[p])
            pltpu.async_copy(tc.at[pl.ds(r40, C // 2)], CB[p], GS[p])

        def wait_gathers(p):
            for dref in (NB[p], DB[p], EB[p]):
                pltpu.make_async_copy(tb.at[pl.ds(0, C)], dref, GS[p]).wait()
            pltpu.make_async_copy(tc.at[pl.ds(0, C // 2)], CB[p],
                                  GS[p]).wait()

        def fire_out(j, p):
            r40 = (s * NCH + j) * (C // 2)
            pltpu.async_copy(CB[p], eno.at[pl.ds(r40, C // 2)], WS[p])
            pltpu.async_copy(NB[p], nacc.at[DS[p].at[0]], SS[p], add=True)
            pltpu.async_copy(EB[p], dacc.at[DS[p].at[0]], SS[p], add=True)

        def wait_out(p):
            pltpu.make_async_copy(CB[p], eno.at[pl.ds(0, C // 2)],
                                  WS[p]).wait()
            pltpu.make_async_copy(NB[p], nmo.at[pl.ds(0, C)], SS[p]).wait()
            pltpu.make_async_copy(EB[p], dno.at[pl.ds(0, C)], SS[p]).wait()

        def compute(p):
            def rowfn(rr, acc):
                out = list(acc)
                r0 = 2 * rr
                for kk in range(8):
                    f = kk % 4
                    row = r0 + (kk // 4)
                    sl = pl.ds(kk * 16, 16)
                    fsl = pl.ds(f * 16, 16)
                    x = CB[p][rr, sl] + DB[p][row, fsl] + EB[p][row, fsl]
                    CB[p][rr, sl] = x
                    sg = 1.0 / (1.0 + jnp.exp(-x))
                    EB[p][row, fsl] = sg
                    NB[p][row, fsl] = sg * NB[p][row, fsl]
                    out[f] = out[f] + x
                    out[4 + f] = out[4 + f] + x * x
                return tuple(out)

            acc = lax.fori_loop(0, C // 2, rowfn,
                                tuple(jnp.zeros((16,), F32) for _ in range(8)))
            for kk in range(4):
                sl = pl.ds(kk * 16, 16)
                statv[0, sl] += acc[kk]
                statv[1, sl] += acc[kk + 4]

        fire_idx_g(0, 0)
        wait_idx_g(0)
        fire_gathers(0, 0)
        fire_idx_g(1, 1)
        fire_idx_s(0, 0)

        def body2(t, carry):
            for p in (0, 1):
                j = t * 2 + p
                q = 1 - p
                wait_gathers(p)

                @pl.when(j + 1 < NCH)
                def _():
                    @pl.when(j >= 1)
                    def _():
                        wait_out(q)
                    fire_idx_s(j + 1, q)
                    wait_idx_g(q)
                    fire_gathers(j + 1, q)

                    @pl.when(j + 2 < NCH)
                    def _():
                        fire_idx_g(j + 2, p)

                compute(p)
                wait_idx_s(p)
                fire_out(j, p)
            return carry

        lax.fori_loop(0, NCH // 2, body2, 0)
        wait_out(0)
        wait_out(1)
        plsc.subcore_barrier()
        done = 0
        while done < 624:
            w = min(C, 624 - done)
            sl = pl.ds(s * 624 + done, w)
            pltpu.sync_copy(nacc.at[sl], nmo.at[sl])
            pltpu.sync_copy(dacc.at[sl], dno.at[sl])
            done += w

        @pl.when(s == NSUB - 1)
        def _():
            sl = pl.ds(16 * 624, N - 16 * 624)
            pltpu.sync_copy(nacc.at[sl], nmo.at[sl])
            pltpu.sync_copy(dacc.at[sl], dno.at[sl])

        pltpu.sync_copy(statv.at[pl.ds(0, 1)], ssum.at[srow + s, pl.ds(0, 1)])
        pltpu.sync_copy(statv.at[pl.ds(1, 1)], ssq.at[srow + s, pl.ds(0, 1)])

    @pl.when(c == 0)
    def _():
        do_half(blo, dlo, elo, clo, enlo, nlo, delo, 0)

    @pl.when(c == 1)
    def _():
        do_half(bhi, dhi, ehi, chi, enhi, nhi, dehi, NSUB)


def _sc_edge(src2, dst2, blo, bhi, dlo, dhi, elo, ehi, clo, chi):
    mesh = plsc.VectorSubcoreMesh(core_axis_name="c", subcore_axis_name="s",
                                  num_cores=2, num_subcores=NSUB)
    out_type = [jax.ShapeDtypeStruct((E // 2, D), F32),
                jax.ShapeDtypeStruct((E // 2, D), F32),
                jax.ShapeDtypeStruct((N, H), F32),
                jax.ShapeDtypeStruct((N, H), F32),
                jax.ShapeDtypeStruct((N, H), F32),
                jax.ShapeDtypeStruct((N, H), F32),
                jax.ShapeDtypeStruct((2 * NSUB, 8, H), F32),
                jax.ShapeDtypeStruct((2 * NSUB, 8, H), F32)]
    scratch = ([pltpu.VMEM((1, C), jnp.int32)] * 6
               + [pltpu.VMEM((C, H), F32)] * 4
               + [pltpu.VMEM((C // 2, D), F32)] * 2
               + [pltpu.VMEM((C, H), F32)] * 2
               + [pltpu.VMEM((2, H), F32),
                  pltpu.VMEM_SHARED((N, H), F32),
                  pltpu.VMEM_SHARED((N, H), F32)]
               + [pltpu.SemaphoreType.DMA] * 10)
    fn = pl.kernel(_sc_edge_body, out_type=out_type, mesh=mesh,
                   scratch_types=scratch,
                   compiler_params=pltpu.CompilerParams(
                       use_tc_tiling_on_sc=False))
    return fn(src2, dst2, blo, bhi, dlo, dhi, elo, ehi, clo, chi)


def kernel(h, e, edge_index, Wh, bh, We, be, Wl, bl, gamma, beta,
           W1, b1, W2, b2, W3, b3):
    src2 = edge_index[0].reshape(E // C, 1, C)
    dst2 = edge_index[1].reshape(E // C, 1, C)
    hcur = _pe(h, Wh, bh)
    row0 = We[0] + be
    eflo = jnp.broadcast_to(jnp.tile(row0[:H], 2), (E // 2, D))
    efhi = jnp.broadcast_to(jnp.tile(row0[H:], 2), (E // 2, D))
    logits = None
    clo, chi = _edgemm(eflo, efhi, Wl[0, 2], bl[0, 2])
    for l in range(L):
        ah, blo, bhi, dlo, dhi, elo, ehi = _node_mm(hcur, Wl[l], bl[l])
        enlo, enhi, nlo, nhi, delo, dehi, ssum, ssq = _sc_edge(
            src2, dst2, blo, bhi, dlo, dhi, elo, ehi, clo, chi)
        if l < L - 1:
            hcur = _hpost(ah, nlo, nhi, delo, dehi, hcur,
                          gamma[l, 0], beta[l, 0])
            eflo, efhi, clo, chi = _edgebnmm(
                enlo, enhi, eflo, efhi, ssum, ssq,
                gamma[l, 1], beta[l, 1], Wl[l + 1, 2], bl[l + 1, 2])
        else:
            logits = _hlast(ah, nlo, nhi, delo, dehi, hcur,
                            gamma[l, 0], beta[l, 0],
                            W1, b1, W2, b2, W3, b3)
    return logits.reshape(NC)
